# baseline (speedup 1.0000x reference)
import jax
import jax.numpy as jnp
from jax.experimental import pallas as pl
from jax.scipy.linalg import block_diag

N = 10000
E = 320000
D = 128
H = 32
C = 10

NP = 10112
NPQ = NP // 4

BRQ = 632


def _proj_body(x_ref, w_ref, o_ref):
    o_ref[...] = jnp.dot(x_ref[...], w_ref[...],
                         preferred_element_type=jnp.float32)


def _mlp1_body(y_ref, a_ref, b1a_ref, w1b_ref, b1b_ref, s1_ref, t1_ref, o_ref):
    u = jnp.maximum(y_ref[...] + jnp.sum(a_ref[...], axis=0) + b1a_ref[...], 0.0)
    z = jnp.dot(u, w1b_ref[...], preferred_element_type=jnp.float32) + b1b_ref[...]
    o_ref[...] = jnp.maximum(z, 0.0) * s1_ref[...] + t1_ref[...]


def _mlp_body(h_ref, a_ref, wa_ref, ba_ref, wb_ref, bb_ref, s_ref, t_ref, o_ref):
    g = h_ref[...] + jnp.sum(a_ref[...], axis=0)
    u = jnp.maximum(jnp.dot(g, wa_ref[...], preferred_element_type=jnp.float32)
                    + ba_ref[...], 0.0)
    z = jnp.dot(u, wb_ref[...], preferred_element_type=jnp.float32) + bb_ref[...]
    o_ref[...] = jnp.maximum(z, 0.0) * s_ref[...] + t_ref[...]


def _head_body(h_ref, a_ref, w3a_ref, b3a_ref, w3b_ref, b3b_ref, s3_ref,
               t3_ref, fcw_ref, fcb_ref, outw_ref, outb_ref, o_ref):
    g = h_ref[...] + jnp.sum(a_ref[...], axis=0)
    u = jnp.maximum(jnp.dot(g, w3a_ref[...], preferred_element_type=jnp.float32)
                    + b3a_ref[...], 0.0)
    z = jnp.dot(u, w3b_ref[...], preferred_element_type=jnp.float32) + b3b_ref[...]
    h3 = jnp.maximum(z, 0.0) * s3_ref[...] + t3_ref[...]
    h4 = jnp.maximum(jnp.dot(h3, fcw_ref[...], preferred_element_type=jnp.float32)
                     + fcb_ref[...], 0.0)
    o_ref[...] = jnp.dot(h4, outw_ref[...], preferred_element_type=jnp.float32) \
        + outb_ref[...]


def _lsm_body(l_ref, o_ref):
    logits = l_ref[...]
    m = jnp.max(logits, axis=-1, keepdims=True)
    lse = m + jnp.log(jnp.sum(jnp.exp(logits - m), axis=-1, keepdims=True))
    o_ref[...] = logits - lse


_vecq = lambda: pl.BlockSpec((1, 4 * H), lambda i: (0, 0))
_matq = lambda: pl.BlockSpec((4 * H, 4 * H), lambda i: (0, 0))
_rowsq = lambda: pl.BlockSpec((BRQ, D), lambda i: (i, 0))
_partsq = lambda: pl.BlockSpec((1, BRQ, D), lambda i: (0, i, 0))


def kernel(x, edge_index, W1a, b1a, W1b, b1b, g1, bt1, m1, v1,
           W2a, b2a, W2b, b2b, g2, bt2, m2, v2,
           W3a, b3a, W3b, b3b, g3, bt3, m3, v3, fcW, fcb, outW, outb):
    f32 = jnp.float32

    src = edge_index[0]
    dst = edge_index[1]
    blk = lambda W: block_diag(W, W, W, W)
    til = lambda v: jnp.tile(v, 4).reshape(1, 4 * H)
    xq = jnp.pad(x, ((0, NP - N), (0, 0))).reshape(NPQ, 4 * D)
    W1aq = block_diag(W1a, W1a, W1a, W1a)

    def bn_coeffs(g, bt, m, v):
        s = g / jnp.sqrt(v + 1e-5)
        return til(s), til(bt - m * s)

    s1, t1 = bn_coeffs(g1, bt1, m1, v1)
    s2, t2 = bn_coeffs(g2, bt2, m2, v2)
    s3, t3 = bn_coeffs(g3, bt3, m3, v3)

    def agg(hq):
        h = hq.reshape(NP, H)[:N]
        a = jax.ops.segment_sum(h[src], dst, num_segments=N)
        a = jnp.pad(a, ((0, NP - N), (0, 0)))
        return a.reshape(1, NPQ, 4 * H)

    y = pl.pallas_call(
        _proj_body,
        grid=(NPQ // BRQ,),
        in_specs=[pl.BlockSpec((BRQ, 4 * D), lambda i: (i, 0)),
                  pl.BlockSpec((4 * D, 4 * H), lambda i: (0, 0))],
        out_specs=pl.BlockSpec((BRQ, D), lambda i: (i, 0)),
        out_shape=jax.ShapeDtypeStruct((NPQ, D), f32),
    )(xq, W1aq)
    a = agg(y)
    h1 = pl.pallas_call(
        _mlp1_body,
        grid=(NPQ // BRQ,),
        in_specs=[_rowsq(), _partsq(), _vecq(), _matq(), _vecq(), _vecq(),
                  _vecq()],
        out_specs=pl.BlockSpec((BRQ, D), lambda i: (i, 0)),
        out_shape=jax.ShapeDtypeStruct((NPQ, D), f32),
    )(y, a, til(b1a), blk(W1b), til(b1b), s1, t1)

    def mid_layer(h, Wa, ba, Wb, bb, s, t):
        a = agg(h)
        return pl.pallas_call(
            _mlp_body,
            grid=(NPQ // BRQ,),
            in_specs=[_rowsq(), _partsq(), _matq(), _vecq(), _matq(), _vecq(),
                      _vecq(), _vecq()],
            out_specs=pl.BlockSpec((BRQ, D), lambda i: (i, 0)),
            out_shape=jax.ShapeDtypeStruct((NPQ, D), f32),
        )(h, a, blk(Wa), til(ba), blk(Wb), til(bb), s, t)

    h2 = mid_layer(h1, W2a, b2a, W2b, b2b, s2, t2)

    a = agg(h2)
    logits_q = pl.pallas_call(
        _head_body,
        grid=(NPQ // BRQ,),
        in_specs=[_rowsq(), _partsq(), _matq(), _vecq(), _matq(), _vecq(),
                  _vecq(), _vecq(), _matq(), _vecq(),
                  pl.BlockSpec((4 * H, 4 * C), lambda i: (0, 0)),
                  pl.BlockSpec((1, 4 * C), lambda i: (0, 0))],
        out_specs=pl.BlockSpec((BRQ, 4 * C), lambda i: (i, 0)),
        out_shape=jax.ShapeDtypeStruct((NPQ, 4 * C), f32),
    )(h2, a, blk(W3a), til(b3a), blk(W3b), til(b3b), s3, t3,
      blk(fcW), til(fcb), block_diag(outW, outW, outW, outW),
      jnp.tile(outb, 4).reshape(1, 4 * C))

    logits = logits_q.reshape(NP, C)
    out = pl.pallas_call(
        _lsm_body,
        grid=(4,),
        in_specs=[pl.BlockSpec((NP // 4, C), lambda i: (i, 0))],
        out_specs=pl.BlockSpec((NP // 4, C), lambda i: (i, 0)),
        out_shape=jax.ShapeDtypeStruct((NP, C), f32),
    )(logits)
    return out[:N]
